# baseline (speedup 1.0000x reference)
import functools

import jax
import jax.numpy as jnp
from jax import lax
from jax.experimental import pallas as pl
from jax.experimental.pallas import tpu as pltpu
from jax.experimental.pallas import tpu_sc as plsc

L = 16


def _entity_pool_sc(table, idx, masks, B, M, E):
    D = table.shape[1]
    BE, K = idx.shape
    KH = K // 2
    info = plsc.get_sparse_core_info()
    nc, ns = info.num_cores, info.num_subcores
    bpc = B // nc
    epw = E // ns

    mesh = plsc.VectorSubcoreMesh(core_axis_name="c", subcore_axis_name="s")

    @functools.partial(
        pl.kernel,
        mesh=mesh,
        out_type=jax.ShapeDtypeStruct((BE, D), jnp.float32),
        scratch_types=[
            pltpu.VMEM_SHARED((M, D), jnp.float32),
            pltpu.VMEM_SHARED((M, D), jnp.float32),
            pltpu.VMEM((bpc, epw, K), jnp.int32),
            pltpu.VMEM((bpc, epw, K), jnp.int32),
            pltpu.VMEM((KH, D), jnp.float32),
            pltpu.VMEM((KH, D), jnp.float32),
            pltpu.VMEM((epw, D), jnp.float32),
            pltpu.SemaphoreType.DMA,
            pltpu.SemaphoreType.DMA,
            pltpu.SemaphoreType.DMA,
        ],
    )
    def run(table_hbm, idx_hbm, mask_hbm, out_hbm,
            shared0, shared1, idx_v, mask_v, buf0, buf1, out_v,
            stage_sem, gsem0, gsem1):
        cid = lax.axis_index("c")
        sid = lax.axis_index("s")
        rows_per_tile = M // ns
        shareds = (shared0, shared1)
        bufs = (buf0, buf1)
        gsems = (gsem0, gsem1)

        def stage_piece(lb):
            src = table_hbm.at[
                pl.ds((cid * bpc + lb) * M + sid * rows_per_tile, rows_per_tile), :]
            dst = shareds[lb % 2].at[pl.ds(sid * rows_per_tile, rows_per_tile), :]
            return pltpu.make_async_copy(src, dst, stage_sem)

        for lb in range(bpc):
            row0 = (cid * bpc + lb) * E + sid * epw
            pltpu.sync_copy(idx_hbm.at[pl.ds(row0, epw), :], idx_v.at[lb])
            pltpu.sync_copy(mask_hbm.at[pl.ds(row0, epw), :], mask_v.at[lb])

        stage_piece(0).start()
        for lb in range(bpc):
            stage_piece(lb).wait()
            plsc.subcore_barrier()
            if lb + 1 < bpc:
                stage_piece(lb + 1).start()
            src_tab = shareds[lb % 2]

            def fetch_half(e, h, lb=lb, src_tab=src_tab):
                iv = idx_v[lb, e, pl.ds(h * KH, L)]
                for j in range(KH):
                    pltpu.make_async_copy(
                        src_tab.at[pl.ds(iv[j], 1), :],
                        bufs[h].at[pl.ds(j, 1), :],
                        gsems[h],
                    ).start()

            def drain_half(h, src_tab=src_tab):
                pltpu.make_async_copy(
                    src_tab.at[pl.ds(0, KH), :], bufs[h], gsems[h]).wait()

            def half_pool(e, h, lb=lb):
                buf = bufs[h]
                mv = mask_v[lb, e, pl.ds(h * KH, L)]
                bv = jnp.where(mv == 0, jnp.float32(-1e30), jnp.float32(0.0))
                splats = [jnp.full((L,), bv[j], dtype=jnp.float32)
                          for j in range(KH)]

                def cbody(c, carry):
                    off = c * L
                    if h == 0:
                        acc = buf[0, pl.ds(off, L)] + splats[0]
                        k0 = 1
                    else:
                        acc = out_v[e, pl.ds(off, L)]
                        k0 = 0
                    for kk in range(k0, KH):
                        acc = jnp.maximum(acc, buf[kk, pl.ds(off, L)] + splats[kk])
                    out_v[e, pl.ds(off, L)] = acc
                    return carry

                lax.fori_loop(0, D // L, cbody, 0)

            fetch_half(0, 0)
            fetch_half(0, 1)

            def ebody(e, carry):
                drain_half(0)
                half_pool(e, 0)

                @pl.when(e + 1 < epw)
                def _():
                    fetch_half(e + 1, 0)

                drain_half(1)
                half_pool(e, 1)

                @pl.when(e + 1 < epw)
                def _():
                    fetch_half(e + 1, 1)

                return carry

            lax.fori_loop(0, epw, ebody, 0)
            row0 = (cid * bpc + lb) * E + sid * epw
            pltpu.sync_copy(out_v, out_hbm.at[pl.ds(row0, epw), :])

    return run(table, idx, masks)


def kernel(mention_reprs, entities, entity_masks):
    B, M, D = mention_reprs.shape
    _, E, K = entities.shape
    table = mention_reprs.reshape(B * M, D)
    idx = entities.reshape(B * E, K)
    masks = entity_masks.reshape(B * E, K)
    out = _entity_pool_sc(table, idx, masks, B, M, E)
    return out.reshape(B, E, D)
